# baseline (speedup 1.0000x reference)
import functools

import jax
import jax.numpy as jnp
from jax import lax
from jax.experimental import pallas as pl
from jax.experimental.pallas import tpu as pltpu
from jax.experimental.pallas import tpu_sc as plsc

N_CLASSES = 1000
N_PER_CLASS = 10
P_TOTAL = N_CLASSES * N_PER_CLASS
C = 384
N = 1024

P_TILE = 2000
GRID = P_TOTAL // P_TILE

NW = 32
B_PAD = 10240
B_PER_W = B_PAD // NW
CHUNK = 80
N_CHUNKS = B_PER_W // CHUNK
IDX_ROWS = B_PAD // CHUNK
ROWS_PER_W = N_CHUNKS
TAIL_W = (P_TOTAL // B_PER_W)
TAIL_ROWS = P_TOTAL - TAIL_W * B_PER_W


def _dist_body(protos_ref, x_ref, sim_ref, dist_ref, idx_ref):
    p = protos_ref[...]
    xm = x_ref[...]
    cross = lax.dot_general(
        p, xm, (((1,), (0,)), ((), ())),
        preferred_element_type=jnp.float32,
    )
    a2 = jnp.sum(xm * xm, axis=0, keepdims=True)
    b2 = jnp.sum(p * p, axis=1, keepdims=True)
    d2 = (a2 + b2) - 2.0 * cross
    d = jnp.sqrt(jnp.maximum(d2, 1e-12))
    dmin = jnp.min(d, axis=1, keepdims=True)
    ii = lax.broadcasted_iota(jnp.int32, d.shape, 1)
    idx = jnp.min(jnp.where(d == dmin, ii, N), axis=1, keepdims=True)
    dist_ref[...] = dmin
    sim_ref[...] = jnp.log((dmin + 1.0) / (dmin + 1e-7))
    idx_ref[...] = idx


_dist_call = pl.pallas_call(
    _dist_body,
    grid=(GRID,),
    in_specs=[
        pl.BlockSpec((P_TILE, C), lambda i: (i, 0)),
        pl.BlockSpec((C, N), lambda i: (0, 0)),
    ],
    out_specs=[
        pl.BlockSpec((P_TILE, 1), lambda i: (i, 0)),
        pl.BlockSpec((P_TILE, 1), lambda i: (i, 0)),
        pl.BlockSpec((P_TILE, 1), lambda i: (i, 0)),
    ],
    out_shape=[
        jax.ShapeDtypeStruct((P_TOTAL, 1), jnp.float32),
        jax.ShapeDtypeStruct((P_TOTAL, 1), jnp.float32),
        jax.ShapeDtypeStruct((P_TOTAL, 1), jnp.int32),
    ],
    compiler_params=pltpu.CompilerParams(
        dimension_semantics=("arbitrary",),
    ),
)


@functools.cache
def _sc_gather_call():
    mesh = plsc.VectorSubcoreMesh(core_axis_name="c", subcore_axis_name="s")

    @functools.partial(
        pl.kernel,
        out_type=jax.ShapeDtypeStruct((P_TOTAL, C), jnp.float32),
        mesh=mesh,
        scratch_types=[
            pltpu.VMEM((ROWS_PER_W, CHUNK), jnp.int32),
            pltpu.VMEM((B_PER_W, C), jnp.float32),
            pltpu.SemaphoreType.DMA,
            pltpu.SemaphoreType.DMA,
        ],
    )
    def _sc_gather(table_hbm, idx_hbm, out_hbm, idx_v, rows_v, gsem, wsem):
        wid = lax.axis_index("s") * 2 + lax.axis_index("c")
        base = wid * B_PER_W
        pltpu.sync_copy(idx_hbm.at[pl.ds(wid * ROWS_PER_W, ROWS_PER_W)], idx_v)
        gathers = [
            pltpu.async_copy(
                table_hbm.at[idx_v.at[j]],
                rows_v.at[pl.ds(j * CHUNK, CHUNK)],
                gsem,
            )
            for j in range(N_CHUNKS)
        ]
        for j in range(N_CHUNKS):
            gathers[j].wait()

            @pl.when((wid != TAIL_W) | (j == 0))
            def _(j=j):
                pltpu.async_copy(
                    rows_v.at[pl.ds(j * CHUNK, CHUNK)],
                    out_hbm.at[pl.ds(base + j * CHUNK, CHUNK)],
                    wsem,
                )

        @pl.when(wid != TAIL_W)
        def _():
            pltpu.make_async_copy(
                rows_v, out_hbm.at[pl.ds(base, B_PER_W)], wsem
            ).wait()

        @pl.when(wid == TAIL_W)
        def _():
            pltpu.make_async_copy(
                rows_v.at[pl.ds(0, CHUNK)],
                out_hbm.at[pl.ds(TAIL_W * B_PER_W, CHUNK)],
                wsem,
            ).wait()

    return _sc_gather


def kernel(x, prototypes):
    xm = x.reshape(C, N)
    protos = prototypes.reshape(P_TOTAL, C)
    sim, dmin, idx = _dist_call(protos, xm)
    xf = xm.T
    idx_pad = jnp.concatenate(
        [idx.reshape(P_TOTAL), jnp.zeros((B_PAD - P_TOTAL,), jnp.int32)]
    ).reshape(IDX_ROWS, CHUNK)
    feats = _sc_gather_call()(xf, idx_pad)
    return (
        sim.reshape(1, P_TOTAL),
        dmin.reshape(1, N_CLASSES, N_PER_CLASS),
        feats.reshape(1, P_TOTAL, C),
    )


# speedup vs baseline: 1.0675x; 1.0675x over previous
import functools

import jax
import jax.numpy as jnp
from jax import lax
from jax.experimental import pallas as pl
from jax.experimental.pallas import tpu as pltpu
from jax.experimental.pallas import tpu_sc as plsc

N_CLASSES = 1000
N_PER_CLASS = 10
P_TOTAL = N_CLASSES * N_PER_CLASS
C = 384
N = 1024

P_TILE = 2000
GRID = P_TOTAL // P_TILE

NW = 32
B_PAD = 10240
B_PER_W = B_PAD // NW
CHUNK = 80
N_CHUNKS = B_PER_W // CHUNK
IDX_ROWS = B_PAD // CHUNK
ROWS_PER_W = N_CHUNKS
TAIL_W = (P_TOTAL // B_PER_W)
TAIL_ROWS = P_TOTAL - TAIL_W * B_PER_W


def _dist_body(protos_ref, x_ref, sim_ref, dist_ref, idx_ref):
    p = protos_ref[...]
    xm = x_ref[...]
    cross = lax.dot_general(
        p, xm, (((1,), (0,)), ((), ())),
        preferred_element_type=jnp.float32,
    )
    a2 = jnp.sum(xm * xm, axis=0, keepdims=True)
    b2 = jnp.sum(p * p, axis=1, keepdims=True)
    d2 = (a2 + b2) - 2.0 * cross
    d = jnp.sqrt(jnp.maximum(d2, 1e-12))
    dmin = jnp.min(d, axis=1, keepdims=True)
    ii = lax.broadcasted_iota(jnp.int32, d.shape, 1)
    idx = jnp.min(jnp.where(d == dmin, ii, N), axis=1, keepdims=True)
    dist_ref[...] = dmin
    sim_ref[...] = jnp.log((dmin + 1.0) / (dmin + 1e-7))
    idx_ref[...] = idx


_dist_call = pl.pallas_call(
    _dist_body,
    grid=(GRID,),
    in_specs=[
        pl.BlockSpec((P_TILE, C), lambda i: (i, 0)),
        pl.BlockSpec((C, N), lambda i: (0, 0)),
    ],
    out_specs=[
        pl.BlockSpec((P_TILE, 1), lambda i: (i, 0)),
        pl.BlockSpec((P_TILE, 1), lambda i: (i, 0)),
        pl.BlockSpec((P_TILE, 1), lambda i: (i, 0)),
    ],
    out_shape=[
        jax.ShapeDtypeStruct((P_TOTAL, 1), jnp.float32),
        jax.ShapeDtypeStruct((P_TOTAL, 1), jnp.float32),
        jax.ShapeDtypeStruct((P_TOTAL, 1), jnp.int32),
    ],
    compiler_params=pltpu.CompilerParams(
        dimension_semantics=("arbitrary",),
    ),
)


@functools.cache
def _sc_gather_call():
    mesh = plsc.VectorSubcoreMesh(core_axis_name="c", subcore_axis_name="s")

    @functools.partial(
        pl.kernel,
        out_type=jax.ShapeDtypeStruct((P_TOTAL, C), jnp.float32),
        mesh=mesh,
        scratch_types=[
            pltpu.VMEM((ROWS_PER_W, CHUNK), jnp.int32),
            pltpu.VMEM((B_PER_W, C), jnp.float32),
            pltpu.SemaphoreType.DMA,
            pltpu.SemaphoreType.DMA,
        ],
    )
    def _sc_gather(table_hbm, idx_hbm, out_hbm, idx_v, rows_v, gsem, wsem):
        wid = lax.axis_index("s") * 2 + lax.axis_index("c")
        base = wid * B_PER_W
        pltpu.sync_copy(idx_hbm.at[pl.ds(wid * ROWS_PER_W, ROWS_PER_W)], idx_v)
        gathers = [
            pltpu.async_copy(
                table_hbm.at[idx_v.at[j]],
                rows_v.at[pl.ds(j * CHUNK, CHUNK)],
                gsem,
            )
            for j in range(N_CHUNKS)
        ]
        for g in gathers:
            g.wait()
        del wsem

        @pl.when(wid != TAIL_W)
        def _():
            pltpu.sync_copy(rows_v, out_hbm.at[pl.ds(base, B_PER_W)])

        @pl.when(wid == TAIL_W)
        def _():
            pltpu.sync_copy(
                rows_v.at[pl.ds(0, TAIL_ROWS)],
                out_hbm.at[pl.ds(TAIL_W * B_PER_W, TAIL_ROWS)],
            )

    return _sc_gather


def kernel(x, prototypes):
    xm = x.reshape(C, N)
    protos = prototypes.reshape(P_TOTAL, C)
    sim, dmin, idx = _dist_call(protos, xm)
    xf = xm.T
    idx_pad = jnp.concatenate(
        [idx.reshape(P_TOTAL), jnp.arange(B_PAD - P_TOTAL, dtype=jnp.int32)]
    ).reshape(IDX_ROWS, CHUNK)
    feats = _sc_gather_call()(xf, idx_pad)
    return (
        sim.reshape(1, P_TOTAL),
        dmin.reshape(1, N_CLASSES, N_PER_CLASS),
        feats.reshape(1, P_TOTAL, C),
    )
